# baseline (speedup 1.0000x reference)
import functools

import jax
import jax.numpy as jnp
from jax import lax
from jax.experimental import pallas as pl
from jax.experimental.pallas import tpu as pltpu
from jax.experimental.pallas import tpu_sc as plsc

_NW = 32
_SPLIT = 1
_OUTER = 8000
_STEPS = _OUTER // 16
_UNROLL = 4
_GWS = [128] * (_OUTER // 128) + ([_OUTER % 128] if _OUTER % 128 else [])


def _sc_body(x0_hbm, x1_hbm, x2_hbm, table_hbm, out_hbm,
             xb0, xb1, idx0, idx1, val0, val1,
             semx0, semx1, semg0, semg1, semo0, semo1,
             *, nchunk):
    slots = -(-nchunk // _NW)
    wid = lax.axis_index("s") * 2 + lax.axis_index("c")
    xb = [xb0, xb1]
    idxb = [idx0, idx1]
    valb = [val0, val1]
    semx = [semx0, semx1]
    semg = [semg0, semg1]
    semo = [semo0, semo1]

    def off_of(g):
        cc = wid + g * _NW
        c = jnp.where(cc < nchunk, cc, cc - nchunk)
        return pl.multiple_of(c * _OUTER, 8)

    def x_copies(g):
        ph = g % 2
        off = off_of(g)
        xcols = [x0_hbm, x1_hbm, x2_hbm]
        return [
            pltpu.make_async_copy(xcols[c].at[pl.ds(off, _OUTER)],
                                  xb[ph].at[c], semx[ph])
            for c in range(3)
        ]

    def gather_copies(g):
        ph = g % 2
        cps = []
        base = 0
        for gw in _GWS:
            cps.append(pltpu.make_async_copy(
                table_hbm.at[idxb[ph].at[pl.ds(base, gw)]],
                valb[ph].at[pl.ds(base, gw)],
                semg[ph],
            ))
            base += gw
        return cps

    def out_copy(g):
        ph = g % 2
        return pltpu.make_async_copy(valb[ph],
                                     out_hbm.at[pl.ds(off_of(g), _OUTER)],
                                     semo[ph])

    def compute(g):
        ph = g % 2
        xr, ir = xb[ph], idxb[ph]

        def one_step(sl):
            a = xr[0, sl]
            b = xr[1, sl]
            k = xr[2, sl]
            ir[sl] = (
                (a << 16)
                | ((b & ~7) << 8)
                | ((b & 7) << 7)
                | ((k & 128) << 3)
                | (k & 127)
            ) & 0xFFFFFF

        def step(s, carry):
            for u in range(_UNROLL):
                one_step(pl.ds(pl.multiple_of(16 * (_UNROLL * s + u), 16), 16))
            return carry

        full = _STEPS // _UNROLL
        lax.fori_loop(0, full, step, 0)
        for t in range(full * _UNROLL, _STEPS):
            one_step(pl.ds(16 * t, 16))

    for cp in x_copies(0):
        cp.start()
    for g in range(slots):
        if g + 1 < slots:
            for cp in x_copies(g + 1):
                cp.start()
        for cp in x_copies(g):
            cp.wait()
        compute(g)
        if g >= 2:
            out_copy(g - 2).wait()
        for cp in gather_copies(g):
            cp.start()
        if g >= 1:
            for cp in gather_copies(g - 1):
                cp.wait()
            out_copy(g - 1).start()
    g_last = slots - 1
    for cp in gather_copies(g_last):
        cp.wait()
    out_copy(g_last).start()
    out_copy(g_last - 1).wait()
    out_copy(g_last).wait()


def kernel(X, probs):
    n = X.shape[0]
    xt = X.astype(jnp.int32).T
    table = (
        probs.reshape(256, 32, 8, 2, 128)
        .transpose(0, 1, 3, 2, 4)
        .reshape(-1)
    )
    rows = n // _SPLIT
    nchunk = rows // _OUTER
    mesh = plsc.VectorSubcoreMesh(core_axis_name="c", subcore_axis_name="s")
    run = functools.partial(
        pl.kernel,
        mesh=mesh,
        out_type=jax.ShapeDtypeStruct((rows,), jnp.float32),
        scratch_types=[
            pltpu.VMEM((3, _OUTER), jnp.int32),
            pltpu.VMEM((3, _OUTER), jnp.int32),
            pltpu.VMEM((_OUTER,), jnp.int32),
            pltpu.VMEM((_OUTER,), jnp.int32),
            pltpu.VMEM((_OUTER,), jnp.float32),
            pltpu.VMEM((_OUTER,), jnp.float32),
            pltpu.SemaphoreType.DMA,
            pltpu.SemaphoreType.DMA,
            pltpu.SemaphoreType.DMA,
            pltpu.SemaphoreType.DMA,
            pltpu.SemaphoreType.DMA,
            pltpu.SemaphoreType.DMA,
        ],
        compiler_params=pltpu.CompilerParams(
            needs_layout_passes=False, use_tc_tiling_on_sc=False
        ),
    )(functools.partial(_sc_body, nchunk=nchunk))
    outs = [
        run(xt[0, k * rows:(k + 1) * rows],
            xt[1, k * rows:(k + 1) * rows],
            xt[2, k * rows:(k + 1) * rows],
            table)
        for k in range(_SPLIT)
    ]
    return jnp.concatenate(outs)


# speedup vs baseline: 1.1631x; 1.1631x over previous
import functools

import jax
import jax.numpy as jnp
from jax import lax
from jax.experimental import pallas as pl
from jax.experimental.pallas import tpu as pltpu
from jax.experimental.pallas import tpu_sc as plsc

_NW = 32
_SPLIT = 1
_OUTER = 8000
_STEPS = _OUTER // 16
_UNROLL = 4
_GWS = [128] * (_OUTER // 128) + ([_OUTER % 128] if _OUTER % 128 else [])


def _sc_body(xt_hbm, table_hbm, out_hbm,
             xb0, xb1, idx0, idx1, val0, val1,
             semx0, semx1, semg0, semg1, semo0, semo1,
             *, nchunk):
    slots = -(-nchunk // _NW)
    wid = lax.axis_index("s") * 2 + lax.axis_index("c")
    xb = [xb0, xb1]
    idxb = [idx0, idx1]
    valb = [val0, val1]
    semx = [semx0, semx1]
    semg = [semg0, semg1]
    semo = [semo0, semo1]

    def off_of(g):
        cc = wid + g * _NW
        c = jnp.where(cc < nchunk, cc, cc - nchunk)
        return pl.multiple_of(c * _OUTER, 8)

    def x_copies(g):
        ph = g % 2
        off = off_of(g)
        return [
            pltpu.make_async_copy(xt_hbm.at[c, pl.ds(off, _OUTER)],
                                  xb[ph].at[c], semx[ph])
            for c in range(3)
        ]

    def gather_copies(g):
        ph = g % 2
        cps = []
        base = 0
        for gw in _GWS:
            cps.append(pltpu.make_async_copy(
                table_hbm.at[idxb[ph].at[pl.ds(base, gw)]],
                valb[ph].at[pl.ds(base, gw)],
                semg[ph],
            ))
            base += gw
        return cps

    def out_copy(g):
        ph = g % 2
        return pltpu.make_async_copy(valb[ph],
                                     out_hbm.at[pl.ds(off_of(g), _OUTER)],
                                     semo[ph])

    def compute(g):
        ph = g % 2
        xr, ir = xb[ph], idxb[ph]

        def one_step(sl):
            a = xr[0, sl]
            b = xr[1, sl]
            k = xr[2, sl]
            ir[sl] = (
                (a << 16)
                | ((b & ~7) << 8)
                | ((b & 7) << 7)
                | ((k & 128) << 3)
                | (k & 127)
            ) & 0xFFFFFF

        def step(s, carry):
            for u in range(_UNROLL):
                one_step(pl.ds(pl.multiple_of(16 * (_UNROLL * s + u), 16), 16))
            return carry

        full = _STEPS // _UNROLL
        lax.fori_loop(0, full, step, 0)
        for t in range(full * _UNROLL, _STEPS):
            one_step(pl.ds(16 * t, 16))

    for cp in x_copies(0):
        cp.start()
    for g in range(slots):
        if g + 1 < slots:
            for cp in x_copies(g + 1):
                cp.start()
        for cp in x_copies(g):
            cp.wait()
        compute(g)
        if g >= 2:
            out_copy(g - 2).wait()
        for cp in gather_copies(g):
            cp.start()
        if g >= 1:
            for cp in gather_copies(g - 1):
                cp.wait()
            out_copy(g - 1).start()
    g_last = slots - 1
    for cp in gather_copies(g_last):
        cp.wait()
    out_copy(g_last).start()
    out_copy(g_last - 1).wait()
    out_copy(g_last).wait()


def kernel(X, probs):
    n = X.shape[0]
    xt = X.astype(jnp.int32).T
    table = (
        probs.reshape(256, 32, 8, 2, 128)
        .transpose(0, 1, 3, 2, 4)
        .reshape(-1)
    )
    rows = n // _SPLIT
    nchunk = rows // _OUTER
    mesh = plsc.VectorSubcoreMesh(core_axis_name="c", subcore_axis_name="s")
    run = functools.partial(
        pl.kernel,
        mesh=mesh,
        out_type=jax.ShapeDtypeStruct((rows,), jnp.float32),
        scratch_types=[
            pltpu.VMEM((3, _OUTER), jnp.int32),
            pltpu.VMEM((3, _OUTER), jnp.int32),
            pltpu.VMEM((_OUTER,), jnp.int32),
            pltpu.VMEM((_OUTER,), jnp.int32),
            pltpu.VMEM((_OUTER,), jnp.float32),
            pltpu.VMEM((_OUTER,), jnp.float32),
            pltpu.SemaphoreType.DMA,
            pltpu.SemaphoreType.DMA,
            pltpu.SemaphoreType.DMA,
            pltpu.SemaphoreType.DMA,
            pltpu.SemaphoreType.DMA,
            pltpu.SemaphoreType.DMA,
        ],
        compiler_params=pltpu.CompilerParams(
            needs_layout_passes=False, use_tc_tiling_on_sc=False
        ),
    )(functools.partial(_sc_body, nchunk=nchunk))
    outs = [run(xt[:, k * rows:(k + 1) * rows], table) for k in range(_SPLIT)]
    return jnp.concatenate(outs)
